# baseline (speedup 1.0000x reference)
import functools

import jax
import jax.numpy as jnp
from jax import lax
from jax.experimental import pallas as pl
from jax.experimental.pallas import tpu as pltpu
from jax.experimental.pallas import tpu_sc as plsc

B, L, D = 16, 512, 256
F = 256
NBINS = 256
T = 4096
TBL = (B + 1) * L
ZROW = B * L
ROWS = B * T

NC, NS = 2, 16
NW = NC * NS
RPW = ROWS // NW
CH = 128
NCH = RPW // CH


def _ln(h, g, b):
    m = jnp.mean(h, axis=-1, keepdims=True)
    v = jnp.mean((h - m) ** 2, axis=-1, keepdims=True)
    return (h - m) / jnp.sqrt(v + 1e-5) * g + b


def _conv3(x, w_ref):
    xb = x.astype(jnp.bfloat16)
    z = jnp.zeros((1, x.shape[-1]), jnp.bfloat16)
    xm = jnp.concatenate([z, xb[:-1, :]], axis=0)
    xp = jnp.concatenate([xb[1:, :], z], axis=0)
    w0 = w_ref[0].astype(jnp.bfloat16)
    w1 = w_ref[1].astype(jnp.bfloat16)
    w2 = w_ref[2].astype(jnp.bfloat16)
    return (jnp.dot(xm, w0, preferred_element_type=jnp.float32)
            + jnp.dot(xb, w1, preferred_element_type=jnp.float32)
            + jnp.dot(xp, w2, preferred_element_type=jnp.float32))


def _vp(x, m1, W1r, b1r, g1r, be1r, W2r, b2r, g2r, be2r, Wlr, blr):
    h = _ln(jax.nn.relu(_conv3(x, W1r) + b1r[...]), g1r[...], be1r[...])
    h = _ln(jax.nn.relu(_conv3(h, W2r) + b2r[...]), g2r[...], be2r[...])
    return (jnp.dot(h, Wlr[...], preferred_element_type=jnp.float32)
            + blr[...]) * m1


def _bucket_embed(t, bins_ref, emb_ref):
    idx = jnp.sum((bins_ref[...] < t).astype(jnp.int32), axis=1,
                  keepdims=True)
    lanes = lax.broadcasted_iota(jnp.int32, (1, NBINS), 1)
    oh = (idx == lanes).astype(jnp.float32)
    return jnp.dot(oh, emb_ref[...], preferred_element_type=jnp.float32)


def _dur_kernel(x_ref, m_ref, W1, b1, g1, be1, W2, b2, g2, be2, Wl, bl,
                pred_ref):
    pred_ref[0] = _vp(x_ref[0], 1.0 - m_ref[0], W1, b1, g1, be1,
                      W2, b2, g2, be2, Wl, bl)


def _pitch_kernel(x_ref, m_ref, t_ref, bins_ref, emb_ref,
                  W1, b1, g1, be1, W2, b2, g2, be2, Wl, bl,
                  pred_ref, x1_ref):
    x = x_ref[0]
    pred_ref[0] = _vp(x, 1.0 - m_ref[0], W1, b1, g1, be1,
                      W2, b2, g2, be2, Wl, bl)
    x1_ref[0] = x + _bucket_embed(t_ref[0], bins_ref, emb_ref)


def _energy_kernel(x_ref, m_ref, t_ref, bins_ref, emb_ref,
                   W1, b1, g1, be1, W2, b2, g2, be2, Wl, bl,
                   pred_ref, x2_ref):
    x = x_ref[0]
    pred_ref[0] = _vp(x, 1.0 - m_ref[0], W1, b1, g1, be1,
                      W2, b2, g2, be2, Wl, bl)
    live = (pl.program_id(0) < B).astype(jnp.float32)
    x2_ref[...] = (x + _bucket_embed(t_ref[0], bins_ref, emb_ref)) * live


def _index_kernel(d_ref, ml_ref, gidx_ref, mel_ref):
    b = pl.program_id(0)
    d = d_ref[0]
    row = lax.broadcasted_iota(jnp.int32, (L, L), 0)
    col = lax.broadcasted_iota(jnp.int32, (L, L), 1)
    cum = jnp.dot(d, (row <= col).astype(jnp.float32),
                  preferred_element_type=jnp.float32)
    mel_ref[0] = cum[:, L - 1:L].astype(jnp.int32)
    ml_f = ml_ref[0, 0].astype(jnp.float32)
    base = b.astype(jnp.float32) * L
    chunks = []
    for tb in range(T // L):
        pos = (lax.broadcasted_iota(jnp.int32, (L, 1), 0)
               .astype(jnp.float32) + float(tb * L))
        idx = jnp.sum((cum <= pos).astype(jnp.float32), axis=1,
                      keepdims=True)
        valid = (idx < float(L)) & (pos < ml_f)
        chunks.append(jnp.where(valid, idx + base, float(ZROW)))
    gidx_ref[0] = jnp.concatenate(chunks, axis=0).astype(jnp.int32)


SLAB = 64


def _expand_kernel(table, gidx, out, idx_all, slab, ob, zrow, ssems, wsems,
                   fsem):
    wid = lax.axis_index("s") * NC + lax.axis_index("c")

    iota16 = lax.iota(jnp.int32, 16)
    NR = CH // 16

    def stage_idx(c, carry):
        pltpu.sync_copy(gidx.at[c].at[wid], idx_all.at[pl.ds(c * CH, CH)])
        return carry

    lax.fori_loop(0, NCH, stage_idx, 0)

    def chunk_lo(c):
        return idx_all[pl.ds(c * CH, 16)][0]

    def slab_half(p):
        return slab.at[pl.ds(pl.multiple_of(p * (SLAB * D), D), SLAB * D)]

    def start_slab(c, p):
        @pl.when(chunk_lo(c) < ZROW)
        def _():
            src = table.at[pl.ds(pl.multiple_of(chunk_lo(c) * D, D),
                                 SLAB * D)]
            pltpu.async_copy(src, slab_half(p), ssems.at[p])

    def drain_writes(p):
        def drn(g, carry2):
            for _ in range(16):
                pltpu.make_async_copy(ob.at[pl.ds(0, D)], out.at[pl.ds(0, D)],
                                      wsems.at[p]).wait()
            return carry2

        lax.fori_loop(0, NR, drn, 0)

    pltpu.sync_copy(table.at[pl.ds(ZROW * D, D)], zrow)
    start_slab(0, 0)

    def do_chunk(c, carry):
        p = jnp.bitwise_and(c, 1)

        @pl.when((c >= 1) & (c + 1 < NCH))
        def _():
            drain_writes(1 - p)

        @pl.when(c + 1 < NCH)
        def _():
            start_slab(c + 1, 1 - p)

        lo = chunk_lo(c)
        live = lo < ZROW

        @pl.when(live)
        def _():
            pltpu.make_async_copy(table.at[pl.ds(0, SLAB * D)], slab_half(p),
                                  ssems.at[p]).wait()

        hi = idx_all[pl.ds(c * CH + CH - 16, 16)][15]
        fits = (hi - lo) < SLAB
        pbase_s = p * (SLAB * D)
        pbase_o = p * (CH * D)
        obase = (c * NW + wid) * (CH * D)

        @pl.when(live & fits)
        def _():
            def grp(g, carry2):
                off = (idx_all[pl.ds(c * CH + g * 16, 16)] - lo) * D + pbase_s
                for j in range(16):
                    src = pl.multiple_of(off[j], D)
                    dst = pl.multiple_of(obase + (g * 16 + j) * D, D)
                    pltpu.async_copy(slab.at[pl.ds(src, D)],
                                     out.at[pl.ds(dst, D)], wsems.at[p])
                return carry2

            lax.fori_loop(0, NR, grp, 0)

        @pl.when(live & jnp.logical_not(fits))
        def _():
            def rd(g, carry2):
                rows = idx_all[pl.ds(c * CH + g * 16, 16)] * D
                for j in range(16):
                    src = pl.multiple_of(rows[j], D)
                    dst = pl.multiple_of((g * 16 + j) * D + pbase_o, D)
                    pltpu.async_copy(table.at[pl.ds(src, D)],
                                     ob.at[pl.ds(dst, D)], fsem)
                return carry2

            lax.fori_loop(0, NR, rd, 0)

            def drn(g, carry2):
                for _ in range(16):
                    pltpu.make_async_copy(table.at[pl.ds(0, D)],
                                          ob.at[pl.ds(0, D)], fsem).wait()
                return carry2

            lax.fori_loop(0, NR, drn, 0)

            def wr(g, carry2):
                for j in range(16):
                    srcd = pl.multiple_of((g * 16 + j) * D + pbase_o, D)
                    dstd = pl.multiple_of(obase + (g * 16 + j) * D, D)
                    pltpu.async_copy(ob.at[pl.ds(srcd, D)],
                                     out.at[pl.ds(dstd, D)], wsems.at[p])
                return carry2

            lax.fori_loop(0, NR, wr, 0)

        @pl.when(jnp.logical_not(live))
        def _():
            def zr(g, carry2):
                for j in range(16):
                    dst = pl.multiple_of(obase + (g * 16 + j) * D, D)
                    pltpu.async_copy(zrow, out.at[pl.ds(dst, D)],
                                     wsems.at[p])
                return carry2

            lax.fori_loop(0, NR, zr, 0)

        return carry

    lax.fori_loop(0, NCH, do_chunk, 0)
    drain_writes(0)
    drain_writes(1)


def _full(a):
    return pl.BlockSpec(a.shape, lambda b: (0,) * a.ndim)


def _vp_specs(p):
    ws = [p["W1"], p["b1"], p["g1"], p["be1"], p["W2"], p["b2"], p["g2"],
          p["be2"], p["Wl"], p["bl"]]
    return ws, [_full(w) for w in ws]


def _prep(p):
    return {
        "W1": p["W1"], "W2": p["W2"],
        "b1": p["b1"].reshape(1, F), "g1": p["g1"].reshape(1, F),
        "be1": p["be1"].reshape(1, F),
        "b2": p["b2"].reshape(1, F), "g2": p["g2"].reshape(1, F),
        "be2": p["be2"].reshape(1, F),
        "Wl": p["Wl"], "bl": p["bl"].reshape(1, 1),
    }


def kernel(x, src_mask, pitch_target, energy_target, duration_target,
           max_len, params):
    maskf = src_mask.astype(jnp.float32).reshape(B, L, 1)
    pt = pitch_target.reshape(B, L, 1)
    et = energy_target.reshape(B, L, 1)
    durf = duration_target.astype(jnp.float32).reshape(B, 1, L)
    mlen = jnp.asarray(max_len, jnp.int32).reshape(1, 1)
    inf = jnp.full((1,), jnp.inf, jnp.float32)
    pbins = jnp.concatenate([params["pitch_bins"].astype(jnp.float32), inf]
                            ).reshape(1, NBINS)
    ebins = jnp.concatenate([params["energy_bins"].astype(jnp.float32), inf]
                            ).reshape(1, NBINS)

    xspec = pl.BlockSpec((1, L, D), lambda b: (b, 0, 0))
    cspec = pl.BlockSpec((1, L, 1), lambda b: (b, 0, 0))

    pp = _prep(params["pitch"])
    pw, pwspecs = _vp_specs(pp)
    pitch_pred3, x1 = pl.pallas_call(
        _pitch_kernel,
        grid=(B,),
        in_specs=[xspec, cspec, cspec, _full(pbins),
                  _full(params["pitch_emb"])] + pwspecs,
        out_specs=[cspec, xspec],
        out_shape=[jax.ShapeDtypeStruct((B, L, 1), jnp.float32),
                   jax.ShapeDtypeStruct((B, L, D), jnp.float32)],
    )(x, maskf, pt, pbins, params["pitch_emb"], *pw)

    gidx, mel3 = pl.pallas_call(
        _index_kernel,
        grid=(B,),
        in_specs=[pl.BlockSpec((1, 1, L), lambda b: (b, 0, 0)),
                  pl.BlockSpec(memory_space=pltpu.SMEM)],
        out_specs=[pl.BlockSpec((1, T, 1), lambda b: (b, 0, 0)),
                   pl.BlockSpec((1, 1, 1), lambda b: (b, 0, 0))],
        out_shape=[jax.ShapeDtypeStruct((B, T, 1), jnp.int32),
                   jax.ShapeDtypeStruct((B, 1, 1), jnp.int32)],
    )(durf, mlen)

    ep = _prep(params["energy"])
    ew, ewspecs = _vp_specs(ep)
    cl = lambda b: (jnp.minimum(b, B - 1), 0, 0)
    energy_pred3, x2p = pl.pallas_call(
        _energy_kernel,
        grid=(B + 1,),
        in_specs=[pl.BlockSpec((1, L, D), cl), pl.BlockSpec((1, L, 1), cl),
                  pl.BlockSpec((1, L, 1), cl), _full(ebins),
                  _full(params["energy_emb"])] + ewspecs,
        out_specs=[pl.BlockSpec((1, L, 1), cl),
                   pl.BlockSpec((L, D), lambda b: (b, 0))],
        out_shape=[jax.ShapeDtypeStruct((B, L, 1), jnp.float32),
                   jax.ShapeDtypeStruct((TBL, D), jnp.float32)],
    )(x1, maskf, et, ebins, params["energy_emb"], *ew)

    expand = pl.kernel(
        _expand_kernel,
        out_type=jax.ShapeDtypeStruct((ROWS * D,), jnp.float32),
        mesh=plsc.VectorSubcoreMesh(core_axis_name="c", subcore_axis_name="s"),
        scratch_types=[pltpu.VMEM((NCH * CH,), jnp.int32),
                       pltpu.VMEM((2 * SLAB * D,), jnp.float32),
                       pltpu.VMEM((2 * CH * D,), jnp.float32),
                       pltpu.VMEM((D,), jnp.float32),
                       pltpu.SemaphoreType.DMA((2,)),
                       pltpu.SemaphoreType.DMA((2,)),
                       pltpu.SemaphoreType.DMA],
        compiler_params=pltpu.CompilerParams(use_tc_tiling_on_sc=False),
    )
    out_flat = expand(x2p.reshape(TBL * D), gidx.reshape(NCH, NW, CH))

    dp = _prep(params["dur"])
    dw, dwspecs = _vp_specs(dp)
    log_dur3 = pl.pallas_call(
        _dur_kernel,
        grid=(B,),
        in_specs=[xspec, cspec] + dwspecs,
        out_specs=cspec,
        out_shape=jax.ShapeDtypeStruct((B, L, 1), jnp.float32),
    )(x, maskf, *dw)

    return (out_flat.reshape(B, T, D),
            pitch_pred3.reshape(B, L),
            energy_pred3.reshape(B, L),
            log_dur3.reshape(B, L),
            duration_target,
            mel3.reshape(B))


# speedup vs baseline: 1.0261x; 1.0261x over previous
import functools

import jax
import jax.numpy as jnp
from jax import lax
from jax.experimental import pallas as pl
from jax.experimental.pallas import tpu as pltpu
from jax.experimental.pallas import tpu_sc as plsc

B, L, D = 16, 512, 256
F = 256
NBINS = 256
T = 4096
TBL = (B + 1) * L
ZROW = B * L
ROWS = B * T

NC, NS = 2, 16
NW = NC * NS
RPW = ROWS // NW
CH = 128
NCH = RPW // CH


def _ln(h, g, b):
    m = jnp.mean(h, axis=-1, keepdims=True)
    v = jnp.mean(h * h, axis=-1, keepdims=True) - m * m
    return (h - m) / jnp.sqrt(v + 1e-5) * g + b


def _conv3(x, w_ref):
    z = jnp.zeros((1, x.shape[-1]), jnp.float32)
    xm = jnp.concatenate([z, x[:-1, :]], axis=0)
    xp = jnp.concatenate([x[1:, :], z], axis=0)
    return (jnp.dot(xm, w_ref[0], preferred_element_type=jnp.float32)
            + jnp.dot(x, w_ref[1], preferred_element_type=jnp.float32)
            + jnp.dot(xp, w_ref[2], preferred_element_type=jnp.float32))


def _vp(x, m1, W1r, b1r, g1r, be1r, W2r, b2r, g2r, be2r, Wlr, blr):
    h = _ln(jax.nn.relu(_conv3(x, W1r) + b1r[...]), g1r[...], be1r[...])
    h = _ln(jax.nn.relu(_conv3(h, W2r) + b2r[...]), g2r[...], be2r[...])
    return (jnp.dot(h, Wlr[...], preferred_element_type=jnp.float32)
            + blr[...]) * m1


def _bucket_embed(t, bins_ref, emb_ref):
    idx = jnp.sum((bins_ref[...] < t).astype(jnp.int32), axis=1,
                  keepdims=True)
    lanes = lax.broadcasted_iota(jnp.int32, (1, NBINS), 1)
    oh = (idx == lanes).astype(jnp.float32)
    return jnp.dot(oh, emb_ref[...], preferred_element_type=jnp.float32)


def _dur_kernel(x_ref, m_ref, W1, b1, g1, be1, W2, b2, g2, be2, Wl, bl,
                pred_ref):
    pred_ref[0] = _vp(x_ref[0], 1.0 - m_ref[0], W1, b1, g1, be1,
                      W2, b2, g2, be2, Wl, bl)


def _pitch_kernel(x_ref, m_ref, t_ref, bins_ref, emb_ref,
                  W1, b1, g1, be1, W2, b2, g2, be2, Wl, bl,
                  pred_ref, x1_ref):
    x = x_ref[0]
    pred_ref[0] = _vp(x, 1.0 - m_ref[0], W1, b1, g1, be1,
                      W2, b2, g2, be2, Wl, bl)
    x1_ref[0] = x + _bucket_embed(t_ref[0], bins_ref, emb_ref)


def _energy_kernel(x_ref, m_ref, t_ref, bins_ref, emb_ref,
                   W1, b1, g1, be1, W2, b2, g2, be2, Wl, bl,
                   pred_ref, x2_ref):
    x = x_ref[0]
    pred_ref[0] = _vp(x, 1.0 - m_ref[0], W1, b1, g1, be1,
                      W2, b2, g2, be2, Wl, bl)
    live = (pl.program_id(0) < B).astype(jnp.float32)
    x2_ref[...] = (x + _bucket_embed(t_ref[0], bins_ref, emb_ref)) * live


def _index_kernel(d_ref, ml_ref, gidx_ref, mel_ref):
    b = pl.program_id(0)
    d = d_ref[0]
    row = lax.broadcasted_iota(jnp.int32, (L, L), 0)
    col = lax.broadcasted_iota(jnp.int32, (L, L), 1)
    cum = jnp.dot(d, (row <= col).astype(jnp.float32),
                  preferred_element_type=jnp.float32)
    mel_ref[0] = cum[:, L - 1:L].astype(jnp.int32)
    ml_f = ml_ref[0, 0].astype(jnp.float32)
    base = b.astype(jnp.float32) * L
    chunks = []
    for tb in range(T // L):
        pos = (lax.broadcasted_iota(jnp.int32, (L, 1), 0)
               .astype(jnp.float32) + float(tb * L))
        idx = jnp.sum((cum <= pos).astype(jnp.float32), axis=1,
                      keepdims=True)
        valid = (idx < float(L)) & (pos < ml_f)
        chunks.append(jnp.where(valid, idx + base, float(ZROW)))
    gidx_ref[0] = jnp.concatenate(chunks, axis=0).astype(jnp.int32)


SLAB = 64


def _expand_kernel(table, gidx, out, idx_all, slab, ob, zrow, ssems, wsems,
                   fsem):
    wid = lax.axis_index("s") * NC + lax.axis_index("c")

    iota16 = lax.iota(jnp.int32, 16)
    NR = CH // 16

    def stage_idx(c, carry):
        pltpu.sync_copy(gidx.at[c].at[wid], idx_all.at[pl.ds(c * CH, CH)])
        return carry

    lax.fori_loop(0, NCH, stage_idx, 0)

    def chunk_lo(c):
        return idx_all[pl.ds(c * CH, 16)][0]

    def slab_half(p):
        return slab.at[pl.ds(pl.multiple_of(p * (SLAB * D), D), SLAB * D)]

    def start_slab(c, p):
        @pl.when(chunk_lo(c) < ZROW)
        def _():
            src = table.at[pl.ds(pl.multiple_of(chunk_lo(c) * D, D),
                                 SLAB * D)]
            pltpu.async_copy(src, slab_half(p), ssems.at[p])

    def drain_writes(p):
        def drn(g, carry2):
            for _ in range(16):
                pltpu.make_async_copy(ob.at[pl.ds(0, D)], out.at[pl.ds(0, D)],
                                      wsems.at[p]).wait()
            return carry2

        lax.fori_loop(0, NR, drn, 0)

    pltpu.sync_copy(table.at[pl.ds(ZROW * D, D)], zrow)
    start_slab(0, 0)

    def do_chunk(c, carry):
        p = jnp.bitwise_and(c, 1)

        @pl.when((c >= 1) & (c + 1 < NCH))
        def _():
            drain_writes(1 - p)

        @pl.when(c + 1 < NCH)
        def _():
            start_slab(c + 1, 1 - p)

        lo = chunk_lo(c)
        live = lo < ZROW

        @pl.when(live)
        def _():
            pltpu.make_async_copy(table.at[pl.ds(0, SLAB * D)], slab_half(p),
                                  ssems.at[p]).wait()

        hi = idx_all[pl.ds(c * CH + CH - 16, 16)][15]
        fits = (hi - lo) < SLAB
        pbase_s = p * (SLAB * D)
        pbase_o = p * (CH * D)
        obase = (c * NW + wid) * (CH * D)

        @pl.when(live & fits)
        def _():
            def grp(g, carry2):
                off = (idx_all[pl.ds(c * CH + g * 16, 16)] - lo) * D + pbase_s
                for j in range(16):
                    src = pl.multiple_of(off[j], D)
                    dst = pl.multiple_of(obase + (g * 16 + j) * D, D)
                    pltpu.async_copy(slab.at[pl.ds(src, D)],
                                     out.at[pl.ds(dst, D)], wsems.at[p])
                return carry2

            lax.fori_loop(0, NR, grp, 0)

        @pl.when(live & jnp.logical_not(fits))
        def _():
            def rd(g, carry2):
                rows = idx_all[pl.ds(c * CH + g * 16, 16)] * D
                for j in range(16):
                    src = pl.multiple_of(rows[j], D)
                    dst = pl.multiple_of((g * 16 + j) * D + pbase_o, D)
                    pltpu.async_copy(table.at[pl.ds(src, D)],
                                     ob.at[pl.ds(dst, D)], fsem)
                return carry2

            lax.fori_loop(0, NR, rd, 0)

            def drn(g, carry2):
                for _ in range(16):
                    pltpu.make_async_copy(table.at[pl.ds(0, D)],
                                          ob.at[pl.ds(0, D)], fsem).wait()
                return carry2

            lax.fori_loop(0, NR, drn, 0)

            def wr(g, carry2):
                for j in range(16):
                    srcd = pl.multiple_of((g * 16 + j) * D + pbase_o, D)
                    dstd = pl.multiple_of(obase + (g * 16 + j) * D, D)
                    pltpu.async_copy(ob.at[pl.ds(srcd, D)],
                                     out.at[pl.ds(dstd, D)], wsems.at[p])
                return carry2

            lax.fori_loop(0, NR, wr, 0)

        @pl.when(jnp.logical_not(live))
        def _():
            def zr(g, carry2):
                for j in range(16):
                    dst = pl.multiple_of(obase + (g * 16 + j) * D, D)
                    pltpu.async_copy(zrow, out.at[pl.ds(dst, D)],
                                     wsems.at[p])
                return carry2

            lax.fori_loop(0, NR, zr, 0)

        return carry

    lax.fori_loop(0, NCH, do_chunk, 0)
    drain_writes(0)
    drain_writes(1)


def _full(a):
    return pl.BlockSpec(a.shape, lambda b: (0,) * a.ndim)


def _vp_specs(p):
    ws = [p["W1"], p["b1"], p["g1"], p["be1"], p["W2"], p["b2"], p["g2"],
          p["be2"], p["Wl"], p["bl"]]
    return ws, [_full(w) for w in ws]


def _prep(p):
    return {
        "W1": p["W1"], "W2": p["W2"],
        "b1": p["b1"].reshape(1, F), "g1": p["g1"].reshape(1, F),
        "be1": p["be1"].reshape(1, F),
        "b2": p["b2"].reshape(1, F), "g2": p["g2"].reshape(1, F),
        "be2": p["be2"].reshape(1, F),
        "Wl": p["Wl"], "bl": p["bl"].reshape(1, 1),
    }


def kernel(x, src_mask, pitch_target, energy_target, duration_target,
           max_len, params):
    maskf = src_mask.astype(jnp.float32).reshape(B, L, 1)
    pt = pitch_target.reshape(B, L, 1)
    et = energy_target.reshape(B, L, 1)
    durf = duration_target.astype(jnp.float32).reshape(B, 1, L)
    mlen = jnp.asarray(max_len, jnp.int32).reshape(1, 1)
    inf = jnp.full((1,), jnp.inf, jnp.float32)
    pbins = jnp.concatenate([params["pitch_bins"].astype(jnp.float32), inf]
                            ).reshape(1, NBINS)
    ebins = jnp.concatenate([params["energy_bins"].astype(jnp.float32), inf]
                            ).reshape(1, NBINS)

    xspec = pl.BlockSpec((1, L, D), lambda b: (b, 0, 0))
    cspec = pl.BlockSpec((1, L, 1), lambda b: (b, 0, 0))

    pp = _prep(params["pitch"])
    pw, pwspecs = _vp_specs(pp)
    pitch_pred3, x1 = pl.pallas_call(
        _pitch_kernel,
        grid=(B,),
        in_specs=[xspec, cspec, cspec, _full(pbins),
                  _full(params["pitch_emb"])] + pwspecs,
        out_specs=[cspec, xspec],
        out_shape=[jax.ShapeDtypeStruct((B, L, 1), jnp.float32),
                   jax.ShapeDtypeStruct((B, L, D), jnp.float32)],
    )(x, maskf, pt, pbins, params["pitch_emb"], *pw)

    gidx, mel3 = pl.pallas_call(
        _index_kernel,
        grid=(B,),
        in_specs=[pl.BlockSpec((1, 1, L), lambda b: (b, 0, 0)),
                  pl.BlockSpec(memory_space=pltpu.SMEM)],
        out_specs=[pl.BlockSpec((1, T, 1), lambda b: (b, 0, 0)),
                   pl.BlockSpec((1, 1, 1), lambda b: (b, 0, 0))],
        out_shape=[jax.ShapeDtypeStruct((B, T, 1), jnp.int32),
                   jax.ShapeDtypeStruct((B, 1, 1), jnp.int32)],
    )(durf, mlen)

    ep = _prep(params["energy"])
    ew, ewspecs = _vp_specs(ep)
    cl = lambda b: (jnp.minimum(b, B - 1), 0, 0)
    energy_pred3, x2p = pl.pallas_call(
        _energy_kernel,
        grid=(B + 1,),
        in_specs=[pl.BlockSpec((1, L, D), cl), pl.BlockSpec((1, L, 1), cl),
                  pl.BlockSpec((1, L, 1), cl), _full(ebins),
                  _full(params["energy_emb"])] + ewspecs,
        out_specs=[pl.BlockSpec((1, L, 1), cl),
                   pl.BlockSpec((L, D), lambda b: (b, 0))],
        out_shape=[jax.ShapeDtypeStruct((B, L, 1), jnp.float32),
                   jax.ShapeDtypeStruct((TBL, D), jnp.float32)],
    )(x1, maskf, et, ebins, params["energy_emb"], *ew)

    expand = pl.kernel(
        _expand_kernel,
        out_type=jax.ShapeDtypeStruct((ROWS * D,), jnp.float32),
        mesh=plsc.VectorSubcoreMesh(core_axis_name="c", subcore_axis_name="s"),
        scratch_types=[pltpu.VMEM((NCH * CH,), jnp.int32),
                       pltpu.VMEM((2 * SLAB * D,), jnp.float32),
                       pltpu.VMEM((2 * CH * D,), jnp.float32),
                       pltpu.VMEM((D,), jnp.float32),
                       pltpu.SemaphoreType.DMA((2,)),
                       pltpu.SemaphoreType.DMA((2,)),
                       pltpu.SemaphoreType.DMA],
        compiler_params=pltpu.CompilerParams(use_tc_tiling_on_sc=False),
    )
    out_flat = expand(x2p.reshape(TBL * D), gidx.reshape(NCH, NW, CH))

    dp = _prep(params["dur"])
    dw, dwspecs = _vp_specs(dp)
    log_dur3 = pl.pallas_call(
        _dur_kernel,
        grid=(B,),
        in_specs=[xspec, cspec] + dwspecs,
        out_specs=cspec,
        out_shape=jax.ShapeDtypeStruct((B, L, 1), jnp.float32),
    )(x, maskf, *dw)

    return (out_flat.reshape(B, T, D),
            pitch_pred3.reshape(B, L),
            energy_pred3.reshape(B, L),
            log_dur3.reshape(B, L),
            duration_target,
            mel3.reshape(B))
